# baseline (speedup 1.0000x reference)
import functools

import jax
import jax.numpy as jnp
from jax import lax
from jax.experimental import pallas as pl
from jax.experimental.pallas import tpu as pltpu
from jax.experimental.pallas import tpu_sc as plsc


def _sc_gather(table, idx2d):
    nr, il = idx2d.shape
    v, d = table.shape
    info = plsc.get_sparse_core_info()
    nw = info.num_cores * info.num_subcores
    rows_per_w = nr // nw

    mesh = plsc.VectorSubcoreMesh(core_axis_name="c", subcore_axis_name="s")

    @functools.partial(
        pl.kernel,
        mesh=mesh,
        out_type=jax.ShapeDtypeStruct((nr * il, d), table.dtype),
        scratch_types=[
            pltpu.VMEM((rows_per_w, il), jnp.int32),
            pltpu.VMEM((rows_per_w * il, d), table.dtype),
            pltpu.SemaphoreType.DMA,
            pltpu.SemaphoreType.DMA,
        ],
    )
    def k(table_hbm, idx_hbm, out_hbm, idx_v, rows_v, gsem, ssem):
        wid = lax.axis_index("s") * info.num_cores + lax.axis_index("c")
        base = wid * rows_per_w
        pltpu.sync_copy(idx_hbm.at[pl.ds(base, rows_per_w)], idx_v)
        gathers = []
        for j in range(rows_per_w):
            gathers.append(
                pltpu.async_copy(
                    table_hbm.at[idx_v.at[j]], rows_v.at[pl.ds(j * il, il)], gsem
                )
            )
        scatters = []
        for j in range(rows_per_w):
            gathers[j].wait()
            scatters.append(
                pltpu.async_copy(
                    rows_v.at[pl.ds(j * il, il)],
                    out_hbm.at[pl.ds((base + j) * il, il)],
                    ssem,
                )
            )
        for s in scatters:
            s.wait()

    return k(table, idx2d)


def _mlp(e, W1, b1, W2, b2, block_m, block_n):
    b, k = e.shape
    h = W1.shape[1]
    v = W2.shape[1]
    nb = b // block_m
    nv = pl.cdiv(v, block_n)

    def body(e_ref, w1_ref, b1_ref, w2_ref, b2_ref, out_ref, h_ref):
        @pl.when(pl.program_id(1) == 0)
        def _():
            h_ref[...] = jnp.tanh(
                jnp.dot(e_ref[...], w1_ref[...], preferred_element_type=jnp.float32)
                + b1_ref[...]
            )

        out_ref[...] = (
            jnp.dot(h_ref[...], w2_ref[...], preferred_element_type=jnp.float32)
            + b2_ref[...]
        )

    return pl.pallas_call(
        body,
        grid=(nb, nv),
        in_specs=[
            pl.BlockSpec((block_m, k), lambda i, j: (i, 0)),
            pl.BlockSpec((k, h), lambda i, j: (0, 0)),
            pl.BlockSpec((1, h), lambda i, j: (0, 0)),
            pl.BlockSpec((h, block_n), lambda i, j: (0, j)),
            pl.BlockSpec((1, block_n), lambda i, j: (0, j)),
        ],
        out_specs=pl.BlockSpec((block_m, block_n), lambda i, j: (i, j)),
        out_shape=jax.ShapeDtypeStruct((b, v), jnp.float32),
        scratch_shapes=[pltpu.VMEM((block_m, h), jnp.float32)],
    )(e, W1, b1, W2, b2)


def kernel(x, embed, W1, b1, W2, b2):
    b, w = x.shape
    v, d = embed.shape
    h = W1.shape[1]
    idx = x.reshape(-1).astype(jnp.int32).reshape(-1, 64)
    rows = _sc_gather(embed, idx)
    e = rows.reshape(b, w * d)
    return _mlp(
        e,
        W1,
        b1.reshape(1, h),
        W2,
        b2.reshape(1, v),
        block_m=b,
        block_n=1024,
    )


# speedup vs baseline: 1.0059x; 1.0059x over previous
import functools

import jax
import jax.numpy as jnp
from jax import lax
from jax.experimental import pallas as pl
from jax.experimental.pallas import tpu as pltpu
from jax.experimental.pallas import tpu_sc as plsc


def _sc_gather(table, idx2d):
    nr, il = idx2d.shape
    v, d = table.shape
    info = plsc.get_sparse_core_info()
    nw = info.num_cores * info.num_subcores
    rows_per_w = nr // nw

    mesh = plsc.VectorSubcoreMesh(core_axis_name="c", subcore_axis_name="s")

    @functools.partial(
        pl.kernel,
        mesh=mesh,
        out_type=jax.ShapeDtypeStruct((nr * il, d), table.dtype),
        scratch_types=[
            pltpu.VMEM((rows_per_w, il), jnp.int32),
            pltpu.VMEM((rows_per_w * il, d), table.dtype),
            pltpu.SemaphoreType.DMA,
            pltpu.SemaphoreType.DMA,
        ],
    )
    def k(table_hbm, idx_hbm, out_hbm, idx_v, rows_v, gsem, ssem):
        wid = lax.axis_index("s") * info.num_cores + lax.axis_index("c")
        base = wid * rows_per_w
        pltpu.sync_copy(idx_hbm.at[pl.ds(base, rows_per_w)], idx_v)
        gathers = []
        for j in range(rows_per_w):
            gathers.append(
                pltpu.async_copy(
                    table_hbm.at[idx_v.at[j]], rows_v.at[pl.ds(j * il, il)], gsem
                )
            )
        scatters = []
        for j in range(rows_per_w):
            gathers[j].wait()
            scatters.append(
                pltpu.async_copy(
                    rows_v.at[pl.ds(j * il, il)],
                    out_hbm.at[pl.ds((base + j) * il, il)],
                    ssem,
                )
            )
        for s in scatters:
            s.wait()

    return k(table, idx2d)


def _mlp(e, W1, b1, W2, b2, block_m, block_n):
    b, k = e.shape
    h = W1.shape[1]
    v = W2.shape[1]
    nb = b // block_m
    nv = pl.cdiv(v, block_n)

    def body(e_ref, w1_ref, b1_ref, w2_ref, b2_ref, out_ref, h_ref):
        @pl.when(pl.program_id(1) == 0)
        def _():
            h_ref[...] = jnp.tanh(
                jnp.dot(e_ref[...], w1_ref[...], preferred_element_type=jnp.float32)
                + b1_ref[...]
            )

        out_ref[...] = (
            jnp.dot(h_ref[...], w2_ref[...], preferred_element_type=jnp.float32)
            + b2_ref[...]
        )

    return pl.pallas_call(
        body,
        grid=(nb, nv),
        in_specs=[
            pl.BlockSpec((block_m, k), lambda i, j: (i, 0)),
            pl.BlockSpec((k, h), lambda i, j: (0, 0)),
            pl.BlockSpec((1, h), lambda i, j: (0, 0)),
            pl.BlockSpec((h, block_n), lambda i, j: (0, j)),
            pl.BlockSpec((1, block_n), lambda i, j: (0, j)),
        ],
        out_specs=pl.BlockSpec((block_m, block_n), lambda i, j: (i, j)),
        out_shape=jax.ShapeDtypeStruct((b, v), jnp.float32),
        scratch_shapes=[pltpu.VMEM((block_m, h), jnp.float32)],
    )(e, W1, b1, W2, b2)


def kernel(x, embed, W1, b1, W2, b2):
    b, w = x.shape
    v, d = embed.shape
    h = W1.shape[1]
    idx = x.reshape(-1).astype(jnp.int32).reshape(-1, 128)
    rows = _sc_gather(embed, idx)
    e = rows.reshape(b, w * d)
    return _mlp(
        e,
        W1,
        b1.reshape(1, h),
        W2,
        b2.reshape(1, v),
        block_m=b,
        block_n=1024,
    )
